# baseline (speedup 1.0000x reference)
import jax
import jax.numpy as jnp
from jax.experimental import pallas as pl
from jax.experimental.pallas import tpu as pltpu

_NUM_EMBED = 1024
_EMBED_DIM = 64
_BLK_W = 4096
_COL_T = 1024


def _vq_body(e_ref, z_ref, out_ref, sse_ref, e2_ref, eq_ref, ecat_ref):
    b = pl.program_id(0)
    w = pl.program_id(1)

    zb = z_ref[0]

    @pl.when((b == 0) & (w == 0))
    def _prep():
        e = e_ref[...]
        e2_ref[...] = e + e
        eq_ref[...] = jnp.sum(e * e, axis=1, keepdims=True)
        e_hi = e.astype(jnp.bfloat16)
        e_mid = (e - e_hi.astype(jnp.float32)).astype(jnp.bfloat16)
        ecat_ref[...] = jnp.concatenate([e_hi, e_mid], axis=1)

    zq = jnp.sum(zb * zb, axis=0, keepdims=True)

    mm2 = jax.lax.dot_general(
        e2_ref[...], zb, (((1,), (0,)), ((), ())),
        preferred_element_type=jnp.float32)
    eq = eq_ref[...]
    rowb = jax.lax.broadcasted_iota(jnp.int32, (8, _COL_T), 0)
    m_parts = []
    idx_parts = []
    for ct in range(_BLK_W // _COL_T):
        c0, c1 = ct * _COL_T, (ct + 1) * _COL_T
        zq_t = zq[:, c0:c1]
        mv = jnp.full((8, _COL_T), jnp.inf, jnp.float32)
        iv = jnp.zeros((8, _COL_T), jnp.int32)
        for i in range(_NUM_EMBED // 8):
            dv = (zq_t + eq[i * 8:(i + 1) * 8, :]) - mm2[i * 8:(i + 1) * 8, c0:c1]
            lt = dv < mv
            mv = jnp.minimum(dv, mv)
            iv = jnp.where(lt, i, iv)
        m_t = jnp.min(mv, axis=0, keepdims=True)
        row_t = iv * 8 + rowb
        idx_parts.append(jnp.min(
            jnp.where(mv == m_t, row_t, _NUM_EMBED), axis=0, keepdims=True))
        m_parts.append(m_t)
    m = jnp.concatenate(m_parts, axis=1)
    idx = jnp.concatenate(idx_parts, axis=1)
    iota = jax.lax.broadcasted_iota(jnp.int16, (_NUM_EMBED, _BLK_W), 0)
    onehot = (iota == idx.astype(jnp.int16)).astype(jnp.bfloat16)

    qs = jax.lax.dot_general(
        ecat_ref[...], onehot, (((0,), (0,)), ((), ())),
        preferred_element_type=jnp.float32)
    q = qs[:_EMBED_DIM] + qs[_EMBED_DIM:]

    out_ref[0] = zb + (q - zb)

    part = jnp.sum(m).reshape(1, 1)

    @pl.when((b == 0) & (w == 0))
    def _init():
        sse_ref[...] = jnp.zeros((1, 1), jnp.float32)

    sse_ref[...] += part


def kernel(z, embed_weight):
    batch, ch, hh, ww = z.shape
    hw = hh * ww
    zr = z.reshape(batch, ch, hw)

    grid = (batch, hw // _BLK_W)
    out, sse = pl.pallas_call(
        _vq_body,
        grid=grid,
        in_specs=[
            pl.BlockSpec((_NUM_EMBED, _EMBED_DIM), lambda b, w: (0, 0)),
            pl.BlockSpec((1, ch, _BLK_W), lambda b, w: (b, 0, w)),
        ],
        out_specs=[
            pl.BlockSpec((1, ch, _BLK_W), lambda b, w: (b, 0, w)),
            pl.BlockSpec((1, 1), lambda b, w: (0, 0)),
        ],
        out_shape=[
            jax.ShapeDtypeStruct((batch, ch, hw), jnp.float32),
            jax.ShapeDtypeStruct((1, 1), jnp.float32),
        ],
        scratch_shapes=[
            pltpu.VMEM((_NUM_EMBED, _EMBED_DIM), jnp.float32),
            pltpu.VMEM((_NUM_EMBED, 1), jnp.float32),
            pltpu.VMEM((_NUM_EMBED, 2 * _EMBED_DIM), jnp.bfloat16),
        ],
    )(embed_weight, zr)

    quantized_st = out.reshape(batch, ch, hh, ww)
    m = sse[0, 0] / z.size
    loss = 0.25 * m + m
    return quantized_st, loss


# speedup vs baseline: 1.2057x; 1.2057x over previous
import jax
import jax.numpy as jnp
from jax.experimental import pallas as pl
from jax.experimental.pallas import tpu as pltpu

_NUM_EMBED = 1024
_EMBED_DIM = 64
_BLK_W = 2048
_COL_T = 1024


def _vq_body(e_ref, z_ref, out_ref, sse_ref, e2_ref, eq_ref, ecat_ref):
    b = pl.program_id(0)
    w = pl.program_id(1)

    zb = z_ref[0]

    @pl.when((b == 0) & (w == 0))
    def _prep():
        e = e_ref[...]
        e2_ref[...] = e + e
        eq_ref[...] = jnp.sum(e * e, axis=1, keepdims=True)
        e_hi = e.astype(jnp.bfloat16)
        e_mid = (e - e_hi.astype(jnp.float32)).astype(jnp.bfloat16)
        ecat_ref[...] = jnp.concatenate([e_hi, e_mid], axis=1)

    zq = jnp.sum(zb * zb, axis=0, keepdims=True)

    mm2 = jax.lax.dot_general(
        e2_ref[...], zb, (((1,), (0,)), ((), ())),
        preferred_element_type=jnp.float32)
    eq = eq_ref[...]
    rowb = jax.lax.broadcasted_iota(jnp.int32, (8, _COL_T), 0)
    m_parts = []
    idx_parts = []
    for ct in range(_BLK_W // _COL_T):
        c0, c1 = ct * _COL_T, (ct + 1) * _COL_T
        zq_t = zq[:, c0:c1]
        mv = jnp.full((8, _COL_T), jnp.inf, jnp.float32)
        iv = jnp.zeros((8, _COL_T), jnp.int32)
        for i in range(_NUM_EMBED // 8):
            dv = (zq_t + eq[i * 8:(i + 1) * 8, :]) - mm2[i * 8:(i + 1) * 8, c0:c1]
            lt = dv < mv
            mv = jnp.minimum(dv, mv)
            iv = jnp.where(lt, i, iv)
        m_t = jnp.min(mv, axis=0, keepdims=True)
        row_t = iv * 8 + rowb
        idx_parts.append(jnp.min(
            jnp.where(mv == m_t, row_t, _NUM_EMBED), axis=0, keepdims=True))
        m_parts.append(m_t)
    m = jnp.concatenate(m_parts, axis=1)
    idx = jnp.concatenate(idx_parts, axis=1)
    iota = jax.lax.broadcasted_iota(jnp.int32, (_NUM_EMBED, _BLK_W), 0)
    onehot = (iota == idx).astype(jnp.bfloat16)

    qs = jax.lax.dot_general(
        ecat_ref[...], onehot, (((0,), (0,)), ((), ())),
        preferred_element_type=jnp.float32)
    q = qs[:_EMBED_DIM] + qs[_EMBED_DIM:]

    out_ref[0] = zb + (q - zb)

    part = jnp.sum(m).reshape(1, 1)

    @pl.when((b == 0) & (w == 0))
    def _init():
        sse_ref[...] = jnp.zeros((1, 1), jnp.float32)

    sse_ref[...] += part


def kernel(z, embed_weight):
    batch, ch, hh, ww = z.shape
    hw = hh * ww
    zr = z.reshape(batch, ch, hw)

    grid = (batch, hw // _BLK_W)
    out, sse = pl.pallas_call(
        _vq_body,
        grid=grid,
        in_specs=[
            pl.BlockSpec((_NUM_EMBED, _EMBED_DIM), lambda b, w: (0, 0)),
            pl.BlockSpec((1, ch, _BLK_W), lambda b, w: (b, 0, w)),
        ],
        out_specs=[
            pl.BlockSpec((1, ch, _BLK_W), lambda b, w: (b, 0, w)),
            pl.BlockSpec((1, 1), lambda b, w: (0, 0)),
        ],
        out_shape=[
            jax.ShapeDtypeStruct((batch, ch, hw), jnp.float32),
            jax.ShapeDtypeStruct((1, 1), jnp.float32),
        ],
        scratch_shapes=[
            pltpu.VMEM((_NUM_EMBED, _EMBED_DIM), jnp.float32),
            pltpu.VMEM((_NUM_EMBED, 1), jnp.float32),
            pltpu.VMEM((_NUM_EMBED, 2 * _EMBED_DIM), jnp.bfloat16),
        ],
    )(embed_weight, zr)

    quantized_st = out.reshape(batch, ch, hh, ww)
    m = sse[0, 0] / z.size
    loss = 0.25 * m + m
    return quantized_st, loss


# speedup vs baseline: 1.2372x; 1.0262x over previous
import jax
import jax.numpy as jnp
from jax.experimental import pallas as pl
from jax.experimental.pallas import tpu as pltpu

_NUM_EMBED = 1024
_EMBED_DIM = 64
_BLK_W = 4096
_COL_T = 1024


def _vq_body(e_ref, z_ref, out_ref, sse_ref, e2_ref, eq_ref, ecat_ref):
    b = pl.program_id(0)
    w = pl.program_id(1)

    zb = z_ref[0]

    @pl.when((b == 0) & (w == 0))
    def _prep():
        e = e_ref[...]
        e2_ref[...] = e + e
        eq_ref[...] = jnp.sum(e * e, axis=1, keepdims=True)
        e_hi = e.astype(jnp.bfloat16)
        e_mid = (e - e_hi.astype(jnp.float32)).astype(jnp.bfloat16)
        ecat_ref[...] = jnp.concatenate([e_hi, e_mid], axis=1)

    zq = jnp.sum(zb * zb, axis=0, keepdims=True)

    mm2 = jax.lax.dot_general(
        e2_ref[...], zb, (((1,), (0,)), ((), ())),
        preferred_element_type=jnp.float32)
    eq = eq_ref[...]
    rowb = jax.lax.broadcasted_iota(jnp.int32, (8, _COL_T), 0)
    m_parts = []
    idx_parts = []
    for ct in range(_BLK_W // _COL_T):
        c0, c1 = ct * _COL_T, (ct + 1) * _COL_T
        zq_t = zq[:, c0:c1]
        mv = jnp.full((8, _COL_T), jnp.inf, jnp.float32)
        iv = jnp.zeros((8, _COL_T), jnp.int32)
        for i in range(_NUM_EMBED // 8):
            dv = (zq_t + eq[i * 8:(i + 1) * 8, :]) - mm2[i * 8:(i + 1) * 8, c0:c1]
            lt = dv < mv
            mv = jnp.minimum(dv, mv)
            iv = jnp.where(lt, i, iv)
        m_t = jnp.min(mv, axis=0, keepdims=True)
        row_t = iv * 8 + rowb
        idx_parts.append(jnp.min(
            jnp.where(mv == m_t, row_t, _NUM_EMBED), axis=0, keepdims=True))
        m_parts.append(m_t)
    m = jnp.concatenate(m_parts, axis=1)
    idx = jnp.concatenate(idx_parts, axis=1)
    iota = jax.lax.broadcasted_iota(jnp.int32, (_NUM_EMBED, _BLK_W), 0)
    onehot = (iota == idx).astype(jnp.bfloat16)

    qs = jax.lax.dot_general(
        ecat_ref[...], onehot, (((0,), (0,)), ((), ())),
        preferred_element_type=jnp.float32)
    q = qs[:_EMBED_DIM] + qs[_EMBED_DIM:]

    out_ref[0] = zb + (q - zb)

    part = jnp.sum(m).reshape(1, 1)

    @pl.when((b == 0) & (w == 0))
    def _init():
        sse_ref[...] = jnp.zeros((1, 1), jnp.float32)

    sse_ref[...] += part


def kernel(z, embed_weight):
    batch, ch, hh, ww = z.shape
    hw = hh * ww
    zr = z.reshape(batch, ch, hw)

    grid = (batch, hw // _BLK_W)
    out, sse = pl.pallas_call(
        _vq_body,
        grid=grid,
        in_specs=[
            pl.BlockSpec((_NUM_EMBED, _EMBED_DIM), lambda b, w: (0, 0)),
            pl.BlockSpec((1, ch, _BLK_W), lambda b, w: (b, 0, w)),
        ],
        out_specs=[
            pl.BlockSpec((1, ch, _BLK_W), lambda b, w: (b, 0, w)),
            pl.BlockSpec((1, 1), lambda b, w: (0, 0)),
        ],
        out_shape=[
            jax.ShapeDtypeStruct((batch, ch, hw), jnp.float32),
            jax.ShapeDtypeStruct((1, 1), jnp.float32),
        ],
        scratch_shapes=[
            pltpu.VMEM((_NUM_EMBED, _EMBED_DIM), jnp.float32),
            pltpu.VMEM((_NUM_EMBED, 1), jnp.float32),
            pltpu.VMEM((_NUM_EMBED, 2 * _EMBED_DIM), jnp.bfloat16),
        ],
    )(embed_weight, zr)

    quantized_st = out.reshape(batch, ch, hh, ww)
    m = sse[0, 0] / z.size
    loss = 0.25 * m + m
    return quantized_st, loss
